# baseline (speedup 1.0000x reference)
import jax
import jax.numpy as jnp
from jax import lax
from jax.experimental import pallas as pl
from jax.experimental.pallas import tpu as pltpu
from jax.experimental.pallas import tpu_sc as plsc

B = 16384
D = 64
PACK = 2 * D
NROWS = 1000000
_INFO = plsc.get_sparse_core_info()
NC = _INFO.num_cores
NS = _INFO.num_subcores
NW = NC * NS
B_PER_W = B // NW
IDX_CHUNK = 128
CHUNKS = B_PER_W // IDX_CHUNK

_TC = 20480


def _combine_body(u_ref, i_ref, c_ref):
    c = jnp.concatenate([u_ref[...].T, i_ref[...].T], axis=1)
    c_ref[...] = pltpu.bitcast(c.astype(jnp.bfloat16), jnp.int32)


def _combine_tables(u_tabT, i_tabT):
    grid = (NROWS + _TC - 1) // _TC
    return pl.pallas_call(
        _combine_body,
        grid=(grid,),
        in_specs=[
            pl.BlockSpec((D, _TC), lambda g: (0, g)),
            pl.BlockSpec((D, _TC), lambda g: (0, g)),
        ],
        out_specs=pl.BlockSpec((_TC // 2, PACK), lambda g: (g, 0)),
        out_shape=jax.ShapeDtypeStruct((NROWS // 2, PACK), jnp.int32),
    )(u_tabT, i_tabT)


def _sc_gather_body(u_ids, i_ids, c_tab, u_out, i_out, idx_v, rows_v, sem):
    wid = lax.axis_index("s") * NC + lax.axis_index("c")
    row0 = wid * CHUNKS
    base = wid * B_PER_W
    for ids, out in ((u_ids, u_out), (i_ids, i_out)):
        pltpu.sync_copy(ids.at[pl.ds(row0, CHUNKS)], idx_v)
        copies = []
        for j in range(CHUNKS):
            copies.append(pltpu.async_copy(
                c_tab.at[idx_v.at[j]],
                rows_v.at[pl.ds(j * IDX_CHUNK, IDX_CHUNK)], sem))
        for c in copies:
            c.wait()
        pltpu.sync_copy(rows_v, out.at[pl.ds(base, B_PER_W)])


def _make_sc_gather():
    mesh = plsc.VectorSubcoreMesh(core_axis_name="c", subcore_axis_name="s")
    return pl.kernel(
        _sc_gather_body,
        mesh=mesh,
        out_type=[
            jax.ShapeDtypeStruct((B, PACK), jnp.int32),
            jax.ShapeDtypeStruct((B, PACK), jnp.int32),
        ],
        scratch_types=[
            pltpu.VMEM((CHUNKS, IDX_CHUNK), jnp.int32),
            pltpu.VMEM((B_PER_W, PACK), jnp.int32),
            pltpu.SemaphoreType.DMA,
        ],
    )


_BB = 2048


def _mlp_body(u_ref, i_ref, pu_ref, pi_ref, w1u_ref, w1i_ref, b1_ref,
              w2_ref, b2_ref, w3_ref, b3_ref, w4_ref, b4_ref, out_ref):
    ug = u_ref[...]
    ig = i_ref[...]
    u = lax.bitcast_convert_type(
        jnp.where(pu_ref[...] > 0, ug & jnp.int32(-65536), ug << 16),
        jnp.float32)
    i = lax.bitcast_convert_type(
        jnp.where(pi_ref[...] > 0, ig & jnp.int32(-65536), ig << 16),
        jnp.float32)
    h = jnp.dot(u, w1u_ref[...], preferred_element_type=jnp.float32)
    h = h + jnp.dot(i, w1i_ref[...], preferred_element_type=jnp.float32)
    h = jnp.maximum(h + b1_ref[...], 0.0)
    h = jnp.maximum(jnp.dot(h, w2_ref[...], preferred_element_type=jnp.float32)
                    + b2_ref[...], 0.0)
    h = jnp.maximum(jnp.dot(h, w3_ref[...], preferred_element_type=jnp.float32)
                    + b3_ref[...], 0.0)
    out_ref[...] = (jnp.sum(h * w4_ref[...], axis=1, keepdims=True)
                    + b4_ref[...])


def _mlp(gu, gi, pu, pi, W1, b1, W2, b2, W3, b3, W4, b4):
    z = jnp.zeros((D, 32), jnp.float32)
    w1u = jnp.concatenate([W1[:, :D].T, z], axis=0)
    w1i = jnp.concatenate([z, W1[:, D:].T], axis=0)
    grid = B // _BB
    full = lambda s: pl.BlockSpec(s, lambda i: (0, 0))
    blk = lambda w: pl.BlockSpec((_BB, w), lambda i: (i, 0))
    out = pl.pallas_call(
        _mlp_body,
        grid=(grid,),
        in_specs=[
            blk(PACK), blk(PACK), blk(1), blk(1),
            full((PACK, 32)), full((PACK, 32)), full((1, 32)),
            full((32, 16)), full((1, 16)),
            full((16, 8)), full((1, 8)),
            full((1, 8)), full((1, 1)),
        ],
        out_specs=pl.BlockSpec((_BB, 1), lambda i: (i, 0)),
        out_shape=jax.ShapeDtypeStruct((B, 1), jnp.float32),
    )(gu, gi, pu, pi, w1u, w1i, b1.reshape(1, 32),
      W2.T, b2.reshape(1, 16), W3.T, b3.reshape(1, 8), W4, b4.reshape(1, 1))
    return out.reshape(-1)


def kernel(user_ids, item_ids, user_table, item_table,
           W1, b1, W2, b2, W3, b3, W4, b4):
    uid = user_ids.astype(jnp.int32)
    iid = item_ids.astype(jnp.int32)
    u_ids = (uid // 2).reshape(NW * CHUNKS, IDX_CHUNK)
    i_ids = (iid // 2).reshape(NW * CHUNKS, IDX_CHUNK)
    pu = (uid % 2).reshape(B, 1)
    pi = (iid % 2).reshape(B, 1)
    c_tab = _combine_tables(user_table.T, item_table.T)
    gu, gi = _make_sc_gather()(u_ids, i_ids, c_tab)
    return _mlp(gu, gi, pu, pi, W1, b1, W2, b2, W3, b3, W4, b4)
